# baseline (speedup 1.0000x reference)
import functools

import jax
import jax.numpy as jnp
from jax import lax
from jax.experimental import pallas as pl
from jax.experimental.pallas import tpu as pltpu
from jax.experimental.pallas import tpu_sc as plsc

N = 10000
E = 320000
BATCHES = 8
N_PAD = 10240
C_REAL = 5000
C_PAD = 5120
E_PAD = 327680
P_IDX = 10100
K = 128
NS = 16
NB = 5
CPT = E_PAD // (NS * K)
RB = 1024
CW = 8


def _zero_fill(zb_v, rows, width):
    zero16 = jnp.zeros((16,), jnp.float32)
    for r in range(rows):
        for j in range(width // 16):
            zb_v[r, pl.ds(j * 16, 16)] = zero16


def _make_sc_layer1():
    rows_per_tile = N_PAD // NS
    c1_rows = N_PAD // NS
    c2_rows = C_PAD // NS
    ZR = 64
    mesh = plsc.VectorSubcoreMesh(core_axis_name="c", subcore_axis_name="s")

    @functools.partial(
        pl.kernel,
        out_type=[
            jax.ShapeDtypeStruct((N_PAD, 64), jnp.float32),
            jax.ShapeDtypeStruct((N_PAD, CW), jnp.float32),
            jax.ShapeDtypeStruct((C_PAD, CW), jnp.float32),
        ],
        mesh=mesh,
        scratch_types=[
            pltpu.VMEM((NB, K), jnp.int32),
            pltpu.VMEM((NB, K), jnp.int32),
            pltpu.VMEM((NB, K), jnp.int32),
            pltpu.VMEM((NB, K, 64), jnp.float32),
            pltpu.VMEM((K, CW), jnp.float32),
            pltpu.VMEM((ZR, 64), jnp.float32),
            pltpu.VMEM_SHARED((N_PAD, 64), jnp.float32),
            pltpu.VMEM_SHARED((N_PAD, CW), jnp.float32),
            pltpu.VMEM_SHARED((C_PAD, CW), jnp.float32),
            pltpu.SemaphoreType.DMA,
            pltpu.SemaphoreType.DMA,
        ],
        compiler_params=pltpu.CompilerParams(use_tc_tiling_on_sc=False),
    )
    def layer1(tab_hbm, src_hbm, dst_hbm, ones_hbm, sums_hbm, cnt1_hbm,
               cnt2_hbm, src_v, dst_v, dst2_v, rows_v, ones_v, zb_v,
               acc_sh, cnt1_sh, cnt2_sh, gsem, ssem):
        c = lax.axis_index("c")
        s = lax.axis_index("s")
        _zero_fill(zb_v, ZR, 64)
        r0 = s * rows_per_tile

        @pl.when(c == 0)
        def _():
            for t in range(rows_per_tile // ZR):
                pltpu.sync_copy(zb_v, acc_sh.at[pl.ds(r0 + t * ZR, ZR)])

        @pl.when(c == 1)
        def _():
            pltpu.sync_copy(ones_hbm, ones_v)
            zsmall = zb_v.at[pl.ds(0, ZR), pl.ds(0, CW)]
            for t in range(c1_rows // ZR):
                pltpu.sync_copy(
                    zsmall, cnt1_sh.at[pl.ds(s * c1_rows + t * ZR, ZR)])
            for t in range(c2_rows // ZR):
                pltpu.sync_copy(
                    zsmall, cnt2_sh.at[pl.ds(s * c2_rows + t * ZR, ZR)])

        plsc.subcore_barrier()

        ngath = jnp.where(c == 0, CPT // NB, 0)
        ncnt = jnp.where(c == 0, 0, CPT // NB)
        base0 = s * CPT

        def gbody(t, carry):
            row0 = base0 + t * NB
            pltpu.sync_copy(src_hbm.at[pl.ds(row0, NB)], src_v)
            pltpu.sync_copy(dst_hbm.at[pl.ds(row0, NB)], dst_v)
            gds = [pltpu.async_copy(tab_hbm.at[src_v.at[b]], rows_v.at[b], gsem)
                   for b in range(NB)]
            sds = []
            for b in range(NB):
                gds[b].wait()
                sds.append(pltpu.async_copy(rows_v.at[b], acc_sh.at[dst_v.at[b]],
                                            ssem, add=True))
            for b in range(NB):
                sds[b].wait()
            return carry

        def cbody(t, carry):
            row0 = base0 + t * NB
            pltpu.sync_copy(dst_hbm.at[pl.ds(row0, NB)], dst_v)
            for b in range(NB):
                for j in range(K // 16):
                    sl = pl.ds(j * 16, 16)
                    dst2_v[b, sl] = lax.shift_right_logical(dst_v[b, sl], 1)
            sds = []
            for b in range(NB):
                sds.append(pltpu.async_copy(ones_v, cnt1_sh.at[dst_v.at[b]],
                                            ssem, add=True))
                sds.append(pltpu.async_copy(ones_v, cnt2_sh.at[dst2_v.at[b]],
                                            ssem, add=True))
            for d in sds:
                d.wait()
            return carry

        lax.fori_loop(0, ngath, gbody, 0)
        lax.fori_loop(0, ncnt, cbody, 0)
        plsc.subcore_barrier()

        @pl.when(c == 0)
        def _():
            pltpu.sync_copy(acc_sh.at[pl.ds(r0, rows_per_tile)],
                            sums_hbm.at[pl.ds(r0, rows_per_tile)])

        @pl.when(c == 1)
        def _():
            pltpu.sync_copy(cnt1_sh.at[pl.ds(s * c1_rows, c1_rows)],
                            cnt1_hbm.at[pl.ds(s * c1_rows, c1_rows)])
            pltpu.sync_copy(cnt2_sh.at[pl.ds(s * c2_rows, c2_rows)],
                            cnt2_hbm.at[pl.ds(s * c2_rows, c2_rows)])

    return layer1


def _make_sc_seg(n_rows, width, shift):
    rows_per_tile = n_rows // NS
    ZR = 64
    mesh = plsc.VectorSubcoreMesh(core_axis_name="c", subcore_axis_name="s")

    @functools.partial(
        pl.kernel,
        out_type=jax.ShapeDtypeStruct((n_rows, width), jnp.float32),
        mesh=mesh,
        scratch_types=[
            pltpu.VMEM((NB, K), jnp.int32),
            pltpu.VMEM((NB, K), jnp.int32),
            pltpu.VMEM((NB, K, width), jnp.float32),
            pltpu.VMEM((ZR, width), jnp.float32),
            pltpu.VMEM_SHARED((n_rows, width), jnp.float32),
            pltpu.SemaphoreType.DMA,
            pltpu.SemaphoreType.DMA,
        ],
        compiler_params=pltpu.CompilerParams(use_tc_tiling_on_sc=False),
    )
    def seg_sum(tab_hbm, src_hbm, dst_hbm, out_hbm, src_v, dst_v, rows_v,
                zb_v, acc_sh, gsem, ssem):
        c = lax.axis_index("c")
        s = lax.axis_index("s")
        _zero_fill(zb_v, ZR, width)
        r0 = s * rows_per_tile
        for t in range(rows_per_tile // ZR):
            pltpu.sync_copy(zb_v, acc_sh.at[pl.ds(r0 + t * ZR, ZR)])
        plsc.subcore_barrier()

        ngath = jnp.where(c == 0, CPT // NB, 0)
        base0 = s * CPT

        def body(t, carry):
            row0 = base0 + t * NB
            pltpu.sync_copy(src_hbm.at[pl.ds(row0, NB)], src_v)
            pltpu.sync_copy(dst_hbm.at[pl.ds(row0, NB)], dst_v)
            if shift:
                for b in range(NB):
                    for j in range(K // 16):
                        sl = pl.ds(j * 16, 16)
                        src_v[b, sl] = lax.shift_right_logical(src_v[b, sl], 1)
                        dst_v[b, sl] = lax.shift_right_logical(dst_v[b, sl], 1)
            gds = [pltpu.async_copy(tab_hbm.at[src_v.at[b]], rows_v.at[b], gsem)
                   for b in range(NB)]
            sds = []
            for b in range(NB):
                gds[b].wait()
                sds.append(pltpu.async_copy(rows_v.at[b], acc_sh.at[dst_v.at[b]],
                                            ssem, add=True))
            for b in range(NB):
                sds[b].wait()
            return carry

        lax.fori_loop(0, ngath, body, 0)
        plsc.subcore_barrier()

        @pl.when(c == 0)
        def _():
            pltpu.sync_copy(acc_sh.at[pl.ds(r0, rows_per_tile)],
                            out_hbm.at[pl.ds(r0, rows_per_tile)])

    return seg_sum


@functools.cache
def _get_sc_layer1():
    return _make_sc_layer1()


@functools.cache
def _get_sc_seg(n_rows, width, shift):
    return _make_sc_seg(n_rows, width, shift)


def _sc_layer1(table, src, dst, ones):
    return _get_sc_layer1()(table, src, dst, ones)


def _sc_seg(table, src, dst, n_rows, width, shift):
    return _get_sc_seg(n_rows, width, shift)(table, src, dst)


def _p1_body(x_ref, wl_ref, wr_ref, y1_ref, z1_ref):
    xb = x_ref[...]
    y1_ref[...] = jnp.dot(xb, wl_ref[...], preferred_element_type=jnp.float32)
    z1_ref[...] = jnp.dot(xb, wr_ref[...], preferred_element_type=jnp.float32)


def _p1(x_pad, W_in_l, W_in_r):
    return pl.pallas_call(
        _p1_body,
        grid=(N_PAD // RB,),
        in_specs=[
            pl.BlockSpec((RB, 128), lambda i: (i, 0)),
            pl.BlockSpec((128, 64), lambda i: (0, 0)),
            pl.BlockSpec((128, 64), lambda i: (0, 0)),
        ],
        out_specs=[
            pl.BlockSpec((RB, 64), lambda i: (i, 0)),
            pl.BlockSpec((RB, 64), lambda i: (i, 0)),
        ],
        out_shape=[
            jax.ShapeDtypeStruct((N_PAD, 64), jnp.float32),
            jax.ShapeDtypeStruct((N_PAD, 64), jnp.float32),
        ],
    )(x_pad, W_in_l, W_in_r)


def _p2_body(s1_ref, c1_ref, z1_ref, bi_ref, whl_ref, whr_ref, y2_ref, z2_ref):
    i = pl.program_id(0)
    cnt = jnp.maximum(c1_ref[:, :1], 1.0)
    h = s1_ref[...] / cnt + z1_ref[...] + bi_ref[...]
    h = jnp.maximum(h, 0.0)
    nrm = jnp.maximum(jnp.sqrt(jnp.sum(h * h, axis=1, keepdims=True)), 1e-12)
    h = h / nrm
    rows = i * RB + lax.broadcasted_iota(jnp.int32, (RB, 1), 0)
    h = jnp.where(rows < N, h, 0.0)
    y2_ref[...] = jnp.dot(h, whl_ref[...], preferred_element_type=jnp.float32)
    z2_ref[...] = jnp.dot(h, whr_ref[...], preferred_element_type=jnp.float32)


def _p2(s1, c1, z1, bi, W_h_l, W_h_r):
    return pl.pallas_call(
        _p2_body,
        grid=(N_PAD // RB,),
        in_specs=[
            pl.BlockSpec((RB, 64), lambda i: (i, 0)),
            pl.BlockSpec((RB, CW), lambda i: (i, 0)),
            pl.BlockSpec((RB, 64), lambda i: (i, 0)),
            pl.BlockSpec((1, 64), lambda i: (0, 0)),
            pl.BlockSpec((64, 64), lambda i: (0, 0)),
            pl.BlockSpec((64, 64), lambda i: (0, 0)),
        ],
        out_specs=[
            pl.BlockSpec((RB, 64), lambda i: (i, 0)),
            pl.BlockSpec((RB, 64), lambda i: (i, 0)),
        ],
        out_shape=[
            jax.ShapeDtypeStruct((N_PAD, 64), jnp.float32),
            jax.ShapeDtypeStruct((N_PAD, 64), jnp.float32),
        ],
    )(s1, c1, z1, bi, W_h_l, W_h_r)


def _p3_body(s2_ref, c1_ref, z2_ref, bh_ref, wol_ref, wor_ref, bo_ref,
             y3_ref, z3_ref):
    i = pl.program_id(0)
    cnt = jnp.maximum(c1_ref[:, :1], 1.0)
    h = jnp.maximum(s2_ref[...] / cnt + z2_ref[...] + bh_ref[...], 0.0)
    nrm = jnp.maximum(jnp.sqrt(jnp.sum(h * h, axis=1, keepdims=True)), 1e-12)
    h = h / nrm
    rows = i * RB + lax.broadcasted_iota(jnp.int32, (RB, 1), 0)
    h = jnp.where(rows < N, h, 0.0)
    rj = lax.broadcasted_iota(jnp.int32, (RB // 2, RB), 0)
    ci = lax.broadcasted_iota(jnp.int32, (RB // 2, RB), 1)
    pair = jnp.where(lax.shift_right_logical(ci, 1) == rj, 0.5, 0.0)
    x2 = jnp.dot(pair, h, preferred_element_type=jnp.float32)
    gc = i * (RB // 2) + lax.broadcasted_iota(jnp.int32, (RB // 2, 1), 0)
    cmask = gc < C_REAL
    y3 = jnp.dot(x2, wol_ref[...], preferred_element_type=jnp.float32)
    y3_ref[...] = jnp.where(cmask, y3, 0.0)
    z3_ref[...] = (jnp.dot(x2, wor_ref[...], preferred_element_type=jnp.float32)
                   + bo_ref[...])


def _p3(s2, c1, z2, bh, Wl3, Wr3, bo):
    return pl.pallas_call(
        _p3_body,
        grid=(N_PAD // RB,),
        in_specs=[
            pl.BlockSpec((RB, 64), lambda i: (i, 0)),
            pl.BlockSpec((RB, CW), lambda i: (i, 0)),
            pl.BlockSpec((RB, 64), lambda i: (i, 0)),
            pl.BlockSpec((1, 64), lambda i: (0, 0)),
            pl.BlockSpec((64, 16), lambda i: (0, 0)),
            pl.BlockSpec((64, 16), lambda i: (0, 0)),
            pl.BlockSpec((1, 16), lambda i: (0, 0)),
        ],
        out_specs=[
            pl.BlockSpec((RB // 2, 16), lambda i: (i, 0)),
            pl.BlockSpec((RB // 2, 16), lambda i: (i, 0)),
        ],
        out_shape=[
            jax.ShapeDtypeStruct((C_PAD, 16), jnp.float32),
            jax.ShapeDtypeStruct((C_PAD, 16), jnp.float32),
        ],
    )(s2, c1, z2, bh, Wl3, Wr3, bo)


def _p4_body(s3_ref, c2_ref, z3_ref, b2_ref, out_ref):
    cnt2 = jnp.maximum(c2_ref[:, :1], 1.0)
    o = s3_ref[...] / cnt2 + z3_ref[...]
    col16 = lax.broadcasted_iota(jnp.int32, (1, 16), 1)
    cm = col16 < 10
    o = jnp.where(cm, o, 0.0)
    nrm = jnp.maximum(jnp.sqrt(jnp.sum(o * o, axis=1, keepdims=True)), 1e-12)
    o = o / nrm
    b2 = b2_ref[...]
    col8 = lax.broadcasted_iota(jnp.int32, (1, BATCHES), 1)
    onehot = jnp.where(b2 == col8, 1.0, 0.0)
    dn = (((0,), (0,)), ((), ()))
    gs = lax.dot_general(onehot, o, dn, preferred_element_type=jnp.float32)
    gcnt = lax.dot_general(onehot, jnp.ones_like(o), dn,
                           preferred_element_type=jnp.float32)
    out = gs / jnp.maximum(gcnt, 1.0)
    neg = jnp.where(cm, out, -1e30)
    mx = jnp.max(neg, axis=1, keepdims=True)
    e = jnp.where(cm, jnp.exp(out - mx), 0.0)
    lse = jnp.log(jnp.sum(e, axis=1, keepdims=True))
    out_ref[...] = out - mx - lse


def _p4(s3, c2, z3, batch2):
    return pl.pallas_call(
        _p4_body,
        grid=(1,),
        in_specs=[
            pl.BlockSpec((C_PAD, 16), lambda i: (0, 0)),
            pl.BlockSpec((C_PAD, CW), lambda i: (0, 0)),
            pl.BlockSpec((C_PAD, 16), lambda i: (0, 0)),
            pl.BlockSpec((C_PAD, 1), lambda i: (0, 0)),
        ],
        out_specs=pl.BlockSpec((BATCHES, 16), lambda i: (0, 0)),
        out_shape=jax.ShapeDtypeStruct((BATCHES, 16), jnp.float32),
    )(s3, c2, z3, batch2)


def kernel(x, edge_index, batch, W_in_l, W_in_r, b_in, W_h_l, W_h_r, b_h,
           W_out_l, W_out_r, b_out):
    pad = jnp.full((E_PAD - E,), P_IDX, jnp.int32)
    src = jnp.concatenate([edge_index[0], pad]).reshape(E_PAD // K, K)
    dst = jnp.concatenate([edge_index[1], pad]).reshape(E_PAD // K, K)
    x_pad = jnp.pad(x, ((0, N_PAD - N), (0, 0)))
    batch2 = jnp.concatenate(
        [batch[0::2], jnp.full((C_PAD - C_REAL,), BATCHES, jnp.int32)]
    ).reshape(C_PAD, 1)
    Wl3 = jnp.pad(W_out_l, ((0, 0), (0, 6)))
    Wr3 = jnp.pad(W_out_r, ((0, 0), (0, 6)))
    bo = jnp.pad(b_out, (0, 6)).reshape(1, 16)
    bi = b_in.reshape(1, 64)
    bh = b_h.reshape(1, 64)

    ones = jnp.ones((K, CW), jnp.float32)
    y1, z1 = _p1(x_pad, W_in_l, W_in_r)
    s1, c1, c2 = _sc_layer1(y1, src, dst, ones)
    y2, z2 = _p2(s1, c1, z1, bi, W_h_l, W_h_r)
    s2 = _sc_seg(y2, src, dst, N_PAD, 64, False)
    y3, z3 = _p3(s2, c1, z2, bh, Wl3, Wr3, bo)
    s3 = _sc_seg(y3, src, dst, C_PAD, 16, True)
    out = _p4(s3, c2, z3, batch2)
    return out[:, :10]


# speedup vs baseline: 1.0666x; 1.0666x over previous
import functools

import jax
import jax.numpy as jnp
from jax import lax
from jax.experimental import pallas as pl
from jax.experimental.pallas import tpu as pltpu
from jax.experimental.pallas import tpu_sc as plsc

N = 10000
E = 320000
BATCHES = 8
N_PAD = 10240
C_REAL = 5000
C_PAD = 5120
E_PAD = 327680
P_IDX = 10100
K = 128
NS = 16
NB = 8
CPT = E_PAD // (NS * K)
RB = 1024
CW = 8


def _zero_fill(zb_v, rows, width):
    zero16 = jnp.zeros((16,), jnp.float32)
    for r in range(rows):
        for j in range(width // 16):
            zb_v[r, pl.ds(j * 16, 16)] = zero16


def _make_sc_layer1():
    rows_per_tile = N_PAD // NS
    c1_rows = N_PAD // NS
    c2_rows = C_PAD // NS
    ZR = 64
    mesh = plsc.VectorSubcoreMesh(core_axis_name="c", subcore_axis_name="s")

    @functools.partial(
        pl.kernel,
        out_type=[
            jax.ShapeDtypeStruct((N_PAD, 64), jnp.float32),
            jax.ShapeDtypeStruct((N_PAD, CW), jnp.float32),
            jax.ShapeDtypeStruct((C_PAD, CW), jnp.float32),
        ],
        mesh=mesh,
        scratch_types=[
            pltpu.VMEM((NB, K), jnp.int32),
            pltpu.VMEM((NB, K), jnp.int32),
            pltpu.VMEM((NB, K), jnp.int32),
            pltpu.VMEM((NB, K, 64), jnp.float32),
            pltpu.VMEM((K, CW), jnp.float32),
            pltpu.VMEM((ZR, 64), jnp.float32),
            pltpu.VMEM_SHARED((N_PAD, 64), jnp.float32),
            pltpu.VMEM_SHARED((N_PAD, CW), jnp.float32),
            pltpu.VMEM_SHARED((C_PAD, CW), jnp.float32),
            pltpu.SemaphoreType.DMA,
            pltpu.SemaphoreType.DMA,
        ],
        compiler_params=pltpu.CompilerParams(use_tc_tiling_on_sc=False),
    )
    def layer1(tab_hbm, src_hbm, dst_hbm, ones_hbm, sums_hbm, cnt1_hbm,
               cnt2_hbm, src_v, dst_v, dst2_v, rows_v, ones_v, zb_v,
               acc_sh, cnt1_sh, cnt2_sh, gsem, ssem):
        c = lax.axis_index("c")
        s = lax.axis_index("s")
        _zero_fill(zb_v, ZR, 64)
        r0 = s * rows_per_tile

        @pl.when(c == 0)
        def _():
            for t in range(rows_per_tile // ZR):
                pltpu.sync_copy(zb_v, acc_sh.at[pl.ds(r0 + t * ZR, ZR)])

        @pl.when(c == 1)
        def _():
            pltpu.sync_copy(ones_hbm, ones_v)
            zsmall = zb_v.at[pl.ds(0, ZR), pl.ds(0, CW)]
            for t in range(c1_rows // ZR):
                pltpu.sync_copy(
                    zsmall, cnt1_sh.at[pl.ds(s * c1_rows + t * ZR, ZR)])
            for t in range(c2_rows // ZR):
                pltpu.sync_copy(
                    zsmall, cnt2_sh.at[pl.ds(s * c2_rows + t * ZR, ZR)])

        plsc.subcore_barrier()

        ngath = jnp.where(c == 0, CPT // NB, 0)
        ncnt = jnp.where(c == 0, 0, CPT // NB)
        base0 = s * CPT

        def gbody(t, carry):
            row0 = base0 + t * NB
            pltpu.sync_copy(src_hbm.at[pl.ds(row0, NB)], src_v)
            pltpu.sync_copy(dst_hbm.at[pl.ds(row0, NB)], dst_v)
            gds = [pltpu.async_copy(tab_hbm.at[src_v.at[b]], rows_v.at[b], gsem)
                   for b in range(NB)]
            sds = []
            for b in range(NB):
                gds[b].wait()
                sds.append(pltpu.async_copy(rows_v.at[b], acc_sh.at[dst_v.at[b]],
                                            ssem, add=True))
            for b in range(NB):
                sds[b].wait()
            return carry

        def cbody(t, carry):
            row0 = base0 + t * NB
            pltpu.sync_copy(dst_hbm.at[pl.ds(row0, NB)], dst_v)
            for b in range(NB):
                for j in range(K // 16):
                    sl = pl.ds(j * 16, 16)
                    dst2_v[b, sl] = lax.shift_right_logical(dst_v[b, sl], 1)
            sds = []
            for b in range(NB):
                sds.append(pltpu.async_copy(ones_v, cnt1_sh.at[dst_v.at[b]],
                                            ssem, add=True))
                sds.append(pltpu.async_copy(ones_v, cnt2_sh.at[dst2_v.at[b]],
                                            ssem, add=True))
            for d in sds:
                d.wait()
            return carry

        lax.fori_loop(0, ngath, gbody, 0)
        lax.fori_loop(0, ncnt, cbody, 0)
        plsc.subcore_barrier()

        @pl.when(c == 0)
        def _():
            pltpu.sync_copy(acc_sh.at[pl.ds(r0, rows_per_tile)],
                            sums_hbm.at[pl.ds(r0, rows_per_tile)])

        @pl.when(c == 1)
        def _():
            pltpu.sync_copy(cnt1_sh.at[pl.ds(s * c1_rows, c1_rows)],
                            cnt1_hbm.at[pl.ds(s * c1_rows, c1_rows)])
            pltpu.sync_copy(cnt2_sh.at[pl.ds(s * c2_rows, c2_rows)],
                            cnt2_hbm.at[pl.ds(s * c2_rows, c2_rows)])

    return layer1


def _make_sc_seg(n_rows, width, shift):
    rows_per_tile = n_rows // NS
    ZR = 64
    mesh = plsc.VectorSubcoreMesh(core_axis_name="c", subcore_axis_name="s")

    @functools.partial(
        pl.kernel,
        out_type=jax.ShapeDtypeStruct((n_rows, width), jnp.float32),
        mesh=mesh,
        scratch_types=[
            pltpu.VMEM((NB, K), jnp.int32),
            pltpu.VMEM((NB, K), jnp.int32),
            pltpu.VMEM((NB, K, width), jnp.float32),
            pltpu.VMEM((ZR, width), jnp.float32),
            pltpu.VMEM_SHARED((n_rows, width), jnp.float32),
            pltpu.SemaphoreType.DMA,
            pltpu.SemaphoreType.DMA,
        ],
        compiler_params=pltpu.CompilerParams(use_tc_tiling_on_sc=False),
    )
    def seg_sum(tab_hbm, src_hbm, dst_hbm, out_hbm, src_v, dst_v, rows_v,
                zb_v, acc_sh, gsem, ssem):
        c = lax.axis_index("c")
        s = lax.axis_index("s")
        _zero_fill(zb_v, ZR, width)
        r0 = s * rows_per_tile
        for t in range(rows_per_tile // ZR):
            pltpu.sync_copy(zb_v, acc_sh.at[pl.ds(r0 + t * ZR, ZR)])
        plsc.subcore_barrier()

        ngath = jnp.where(c == 0, CPT // NB, 0)
        base0 = s * CPT

        def body(t, carry):
            row0 = base0 + t * NB
            pltpu.sync_copy(src_hbm.at[pl.ds(row0, NB)], src_v)
            pltpu.sync_copy(dst_hbm.at[pl.ds(row0, NB)], dst_v)
            if shift:
                for b in range(NB):
                    for j in range(K // 16):
                        sl = pl.ds(j * 16, 16)
                        src_v[b, sl] = lax.shift_right_logical(src_v[b, sl], 1)
                        dst_v[b, sl] = lax.shift_right_logical(dst_v[b, sl], 1)
            gds = [pltpu.async_copy(tab_hbm.at[src_v.at[b]], rows_v.at[b], gsem)
                   for b in range(NB)]
            sds = []
            for b in range(NB):
                gds[b].wait()
                sds.append(pltpu.async_copy(rows_v.at[b], acc_sh.at[dst_v.at[b]],
                                            ssem, add=True))
            for b in range(NB):
                sds[b].wait()
            return carry

        lax.fori_loop(0, ngath, body, 0)
        plsc.subcore_barrier()

        @pl.when(c == 0)
        def _():
            pltpu.sync_copy(acc_sh.at[pl.ds(r0, rows_per_tile)],
                            out_hbm.at[pl.ds(r0, rows_per_tile)])

    return seg_sum


@functools.cache
def _get_sc_layer1():
    return _make_sc_layer1()


@functools.cache
def _get_sc_seg(n_rows, width, shift):
    return _make_sc_seg(n_rows, width, shift)


def _sc_layer1(table, src, dst, ones):
    return _get_sc_layer1()(table, src, dst, ones)


def _sc_seg(table, src, dst, n_rows, width, shift):
    return _get_sc_seg(n_rows, width, shift)(table, src, dst)


def _p1_body(x_ref, wl_ref, wr_ref, y1_ref, z1_ref):
    xb = x_ref[...]
    y1_ref[...] = jnp.dot(xb, wl_ref[...], preferred_element_type=jnp.float32)
    z1_ref[...] = jnp.dot(xb, wr_ref[...], preferred_element_type=jnp.float32)


def _p1(x_pad, W_in_l, W_in_r):
    return pl.pallas_call(
        _p1_body,
        grid=(N_PAD // RB,),
        in_specs=[
            pl.BlockSpec((RB, 128), lambda i: (i, 0)),
            pl.BlockSpec((128, 64), lambda i: (0, 0)),
            pl.BlockSpec((128, 64), lambda i: (0, 0)),
        ],
        out_specs=[
            pl.BlockSpec((RB, 64), lambda i: (i, 0)),
            pl.BlockSpec((RB, 64), lambda i: (i, 0)),
        ],
        out_shape=[
            jax.ShapeDtypeStruct((N_PAD, 64), jnp.float32),
            jax.ShapeDtypeStruct((N_PAD, 64), jnp.float32),
        ],
    )(x_pad, W_in_l, W_in_r)


def _p2_body(s1_ref, c1_ref, z1_ref, bi_ref, whl_ref, whr_ref, y2_ref, z2_ref):
    i = pl.program_id(0)
    cnt = jnp.maximum(c1_ref[:, :1], 1.0)
    h = s1_ref[...] / cnt + z1_ref[...] + bi_ref[...]
    h = jnp.maximum(h, 0.0)
    nrm = jnp.maximum(jnp.sqrt(jnp.sum(h * h, axis=1, keepdims=True)), 1e-12)
    h = h / nrm
    rows = i * RB + lax.broadcasted_iota(jnp.int32, (RB, 1), 0)
    h = jnp.where(rows < N, h, 0.0)
    y2_ref[...] = jnp.dot(h, whl_ref[...], preferred_element_type=jnp.float32)
    z2_ref[...] = jnp.dot(h, whr_ref[...], preferred_element_type=jnp.float32)


def _p2(s1, c1, z1, bi, W_h_l, W_h_r):
    return pl.pallas_call(
        _p2_body,
        grid=(N_PAD // RB,),
        in_specs=[
            pl.BlockSpec((RB, 64), lambda i: (i, 0)),
            pl.BlockSpec((RB, CW), lambda i: (i, 0)),
            pl.BlockSpec((RB, 64), lambda i: (i, 0)),
            pl.BlockSpec((1, 64), lambda i: (0, 0)),
            pl.BlockSpec((64, 64), lambda i: (0, 0)),
            pl.BlockSpec((64, 64), lambda i: (0, 0)),
        ],
        out_specs=[
            pl.BlockSpec((RB, 64), lambda i: (i, 0)),
            pl.BlockSpec((RB, 64), lambda i: (i, 0)),
        ],
        out_shape=[
            jax.ShapeDtypeStruct((N_PAD, 64), jnp.float32),
            jax.ShapeDtypeStruct((N_PAD, 64), jnp.float32),
        ],
    )(s1, c1, z1, bi, W_h_l, W_h_r)


def _p3_body(s2_ref, c1_ref, z2_ref, bh_ref, wol_ref, wor_ref, bo_ref,
             y3_ref, z3_ref):
    i = pl.program_id(0)
    cnt = jnp.maximum(c1_ref[:, :1], 1.0)
    h = jnp.maximum(s2_ref[...] / cnt + z2_ref[...] + bh_ref[...], 0.0)
    nrm = jnp.maximum(jnp.sqrt(jnp.sum(h * h, axis=1, keepdims=True)), 1e-12)
    h = h / nrm
    rows = i * RB + lax.broadcasted_iota(jnp.int32, (RB, 1), 0)
    h = jnp.where(rows < N, h, 0.0)
    rj = lax.broadcasted_iota(jnp.int32, (RB // 2, RB), 0)
    ci = lax.broadcasted_iota(jnp.int32, (RB // 2, RB), 1)
    pair = jnp.where(lax.shift_right_logical(ci, 1) == rj, 0.5, 0.0)
    x2 = jnp.dot(pair, h, preferred_element_type=jnp.float32)
    gc = i * (RB // 2) + lax.broadcasted_iota(jnp.int32, (RB // 2, 1), 0)
    cmask = gc < C_REAL
    y3 = jnp.dot(x2, wol_ref[...], preferred_element_type=jnp.float32)
    y3_ref[...] = jnp.where(cmask, y3, 0.0)
    z3_ref[...] = (jnp.dot(x2, wor_ref[...], preferred_element_type=jnp.float32)
                   + bo_ref[...])


def _p3(s2, c1, z2, bh, Wl3, Wr3, bo):
    return pl.pallas_call(
        _p3_body,
        grid=(N_PAD // RB,),
        in_specs=[
            pl.BlockSpec((RB, 64), lambda i: (i, 0)),
            pl.BlockSpec((RB, CW), lambda i: (i, 0)),
            pl.BlockSpec((RB, 64), lambda i: (i, 0)),
            pl.BlockSpec((1, 64), lambda i: (0, 0)),
            pl.BlockSpec((64, 16), lambda i: (0, 0)),
            pl.BlockSpec((64, 16), lambda i: (0, 0)),
            pl.BlockSpec((1, 16), lambda i: (0, 0)),
        ],
        out_specs=[
            pl.BlockSpec((RB // 2, 16), lambda i: (i, 0)),
            pl.BlockSpec((RB // 2, 16), lambda i: (i, 0)),
        ],
        out_shape=[
            jax.ShapeDtypeStruct((C_PAD, 16), jnp.float32),
            jax.ShapeDtypeStruct((C_PAD, 16), jnp.float32),
        ],
    )(s2, c1, z2, bh, Wl3, Wr3, bo)


def _p4_body(s3_ref, c2_ref, z3_ref, b2_ref, out_ref):
    cnt2 = jnp.maximum(c2_ref[:, :1], 1.0)
    o = s3_ref[...] / cnt2 + z3_ref[...]
    col16 = lax.broadcasted_iota(jnp.int32, (1, 16), 1)
    cm = col16 < 10
    o = jnp.where(cm, o, 0.0)
    nrm = jnp.maximum(jnp.sqrt(jnp.sum(o * o, axis=1, keepdims=True)), 1e-12)
    o = o / nrm
    b2 = b2_ref[...]
    col8 = lax.broadcasted_iota(jnp.int32, (1, BATCHES), 1)
    onehot = jnp.where(b2 == col8, 1.0, 0.0)
    dn = (((0,), (0,)), ((), ()))
    gs = lax.dot_general(onehot, o, dn, preferred_element_type=jnp.float32)
    gcnt = lax.dot_general(onehot, jnp.ones_like(o), dn,
                           preferred_element_type=jnp.float32)
    out = gs / jnp.maximum(gcnt, 1.0)
    neg = jnp.where(cm, out, -1e30)
    mx = jnp.max(neg, axis=1, keepdims=True)
    e = jnp.where(cm, jnp.exp(out - mx), 0.0)
    lse = jnp.log(jnp.sum(e, axis=1, keepdims=True))
    out_ref[...] = out - mx - lse


def _p4(s3, c2, z3, batch2):
    return pl.pallas_call(
        _p4_body,
        grid=(1,),
        in_specs=[
            pl.BlockSpec((C_PAD, 16), lambda i: (0, 0)),
            pl.BlockSpec((C_PAD, CW), lambda i: (0, 0)),
            pl.BlockSpec((C_PAD, 16), lambda i: (0, 0)),
            pl.BlockSpec((C_PAD, 1), lambda i: (0, 0)),
        ],
        out_specs=pl.BlockSpec((BATCHES, 16), lambda i: (0, 0)),
        out_shape=jax.ShapeDtypeStruct((BATCHES, 16), jnp.float32),
    )(s3, c2, z3, batch2)


def kernel(x, edge_index, batch, W_in_l, W_in_r, b_in, W_h_l, W_h_r, b_h,
           W_out_l, W_out_r, b_out):
    pad = jnp.full((E_PAD - E,), P_IDX, jnp.int32)
    src = jnp.concatenate([edge_index[0], pad]).reshape(E_PAD // K, K)
    dst = jnp.concatenate([edge_index[1], pad]).reshape(E_PAD // K, K)
    x_pad = jnp.pad(x, ((0, N_PAD - N), (0, 0)))
    batch2 = jnp.concatenate(
        [batch[0::2], jnp.full((C_PAD - C_REAL,), BATCHES, jnp.int32)]
    ).reshape(C_PAD, 1)
    Wl3 = jnp.pad(W_out_l, ((0, 0), (0, 6)))
    Wr3 = jnp.pad(W_out_r, ((0, 0), (0, 6)))
    bo = jnp.pad(b_out, (0, 6)).reshape(1, 16)
    bi = b_in.reshape(1, 64)
    bh = b_h.reshape(1, 64)

    ones = jnp.ones((K, CW), jnp.float32)
    y1, z1 = _p1(x_pad, W_in_l, W_in_r)
    s1, c1, c2 = _sc_layer1(y1, src, dst, ones)
    y2, z2 = _p2(s1, c1, z1, bi, W_h_l, W_h_r)
    s2 = _sc_seg(y2, src, dst, N_PAD, 64, False)
    y3, z3 = _p3(s2, c1, z2, bh, Wl3, Wr3, bo)
    s3 = _sc_seg(y3, src, dst, C_PAD, 16, True)
    out = _p4(s3, c2, z3, batch2)
    return out[:, :10]


# speedup vs baseline: 1.1278x; 1.0574x over previous
import functools

import jax
import jax.numpy as jnp
from jax import lax
from jax.experimental import pallas as pl
from jax.experimental.pallas import tpu as pltpu
from jax.experimental.pallas import tpu_sc as plsc

N = 10000
E = 320000
BATCHES = 8
N_PAD = 10240
C_REAL = 5000
C_PAD = 5120
E_PAD = 327680
P_IDX = 10100
K = 128
NS = 16
NB = 8
CPT = E_PAD // (NS * K)
RB = 1024
CW = 8


def _zero_fill(zb_v, rows, width):
    zero16 = jnp.zeros((16,), jnp.float32)
    for r in range(rows):
        for j in range(width // 16):
            zb_v[r, pl.ds(j * 16, 16)] = zero16


def _make_sc_layer1():
    rows_per_tile = N_PAD // NS
    c1_rows = N_PAD // NS
    c2_rows = C_PAD // NS
    ZR = 64
    mesh = plsc.VectorSubcoreMesh(core_axis_name="c", subcore_axis_name="s")

    @functools.partial(
        pl.kernel,
        out_type=[
            jax.ShapeDtypeStruct((N_PAD, 64), jnp.float32),
            jax.ShapeDtypeStruct((N_PAD, CW), jnp.float32),
            jax.ShapeDtypeStruct((C_PAD, CW), jnp.float32),
        ],
        mesh=mesh,
        scratch_types=[
            pltpu.VMEM((NB, K), jnp.int32),
            pltpu.VMEM((NB, K), jnp.int32),
            pltpu.VMEM((NB, K), jnp.int32),
            pltpu.VMEM((NB, K, 64), jnp.float32),
            pltpu.VMEM((K, CW), jnp.float32),
            pltpu.VMEM((ZR, 64), jnp.float32),
            pltpu.VMEM_SHARED((N_PAD, 64), jnp.float32),
            pltpu.VMEM_SHARED((N_PAD, CW), jnp.float32),
            pltpu.VMEM_SHARED((C_PAD, CW), jnp.float32),
            pltpu.SemaphoreType.DMA,
            pltpu.SemaphoreType.DMA,
        ],
        compiler_params=pltpu.CompilerParams(use_tc_tiling_on_sc=False),
    )
    def layer1(tab_hbm, src_hbm, dst_hbm, ones_hbm, sums_hbm, cnt1_hbm,
               cnt2_hbm, src_v, dst_v, dst2_v, rows_v, ones_v, zb_v,
               acc_sh, cnt1_sh, cnt2_sh, gsem, ssem):
        c = lax.axis_index("c")
        s = lax.axis_index("s")
        _zero_fill(zb_v, ZR, 64)
        r0 = s * rows_per_tile

        @pl.when(c == 0)
        def _():
            for t in range(rows_per_tile // ZR):
                pltpu.sync_copy(zb_v, acc_sh.at[pl.ds(r0 + t * ZR, ZR)])

        @pl.when(c == 1)
        def _():
            pltpu.sync_copy(ones_hbm, ones_v)
            zsmall = zb_v.at[pl.ds(0, ZR), pl.ds(0, CW)]
            for t in range(c1_rows // ZR):
                pltpu.sync_copy(
                    zsmall, cnt1_sh.at[pl.ds(s * c1_rows + t * ZR, ZR)])
            for t in range(c2_rows // ZR):
                pltpu.sync_copy(
                    zsmall, cnt2_sh.at[pl.ds(s * c2_rows + t * ZR, ZR)])

        plsc.subcore_barrier()

        ngath = jnp.where(c == 0, CPT // NB, 0)
        ncnt = jnp.where(c == 0, 0, CPT // NB)
        base0 = s * CPT

        def gbody(t, carry):
            row0 = base0 + t * NB
            pltpu.sync_copy(src_hbm.at[pl.ds(row0, NB)], src_v)
            pltpu.sync_copy(dst_hbm.at[pl.ds(row0, NB)], dst_v)
            gds = [pltpu.async_copy(tab_hbm.at[src_v.at[b]], rows_v.at[b], gsem)
                   for b in range(NB)]
            sds = []
            for b in range(NB):
                gds[b].wait()
                sds.append(pltpu.async_copy(rows_v.at[b], acc_sh.at[dst_v.at[b]],
                                            ssem, add=True))
            for b in range(NB):
                sds[b].wait()
            return carry

        def cbody(t, carry):
            row0 = base0 + t * NB
            pltpu.sync_copy(dst_hbm.at[pl.ds(row0, NB)], dst_v)
            for b in range(NB):
                for j in range(K // 16):
                    sl = pl.ds(j * 16, 16)
                    dst2_v[b, sl] = lax.shift_right_logical(dst_v[b, sl], 1)
            sds = []
            for b in range(NB):
                sds.append(pltpu.async_copy(ones_v, cnt1_sh.at[dst_v.at[b]],
                                            ssem, add=True))
                sds.append(pltpu.async_copy(ones_v, cnt2_sh.at[dst2_v.at[b]],
                                            ssem, add=True))
            for d in sds:
                d.wait()
            return carry

        lax.fori_loop(0, ngath, gbody, 0)
        lax.fori_loop(0, ncnt, cbody, 0)
        plsc.subcore_barrier()

        @pl.when(c == 0)
        def _():
            pltpu.sync_copy(acc_sh.at[pl.ds(r0, rows_per_tile)],
                            sums_hbm.at[pl.ds(r0, rows_per_tile)])

        @pl.when(c == 1)
        def _():
            pltpu.sync_copy(cnt1_sh.at[pl.ds(s * c1_rows, c1_rows)],
                            cnt1_hbm.at[pl.ds(s * c1_rows, c1_rows)])
            pltpu.sync_copy(cnt2_sh.at[pl.ds(s * c2_rows, c2_rows)],
                            cnt2_hbm.at[pl.ds(s * c2_rows, c2_rows)])

    return layer1


def _make_sc_seg(n_rows, width, shift, ch_fast, ch_slow):
    rows_per_tile = n_rows // NS
    assert ch_fast % NB == 0 and ch_slow % NB == 0
    assert ch_fast + ch_slow == CPT
    split = ch_slow > 0
    out_rows = (2 if split else 1) * n_rows
    ZR = 64
    mesh = plsc.VectorSubcoreMesh(core_axis_name="c", subcore_axis_name="s")

    @functools.partial(
        pl.kernel,
        out_type=jax.ShapeDtypeStruct((out_rows, width), jnp.float32),
        mesh=mesh,
        scratch_types=[
            pltpu.VMEM((NB, K), jnp.int32),
            pltpu.VMEM((NB, K), jnp.int32),
            pltpu.VMEM((NB, K, width), jnp.float32),
            pltpu.VMEM((ZR, width), jnp.float32),
            pltpu.VMEM_SHARED((n_rows, width), jnp.float32),
            pltpu.SemaphoreType.DMA,
            pltpu.SemaphoreType.DMA,
        ],
        compiler_params=pltpu.CompilerParams(use_tc_tiling_on_sc=False),
    )
    def seg_sum(tab_hbm, src_hbm, dst_hbm, out_hbm, src_v, dst_v, rows_v,
                zb_v, acc_sh, gsem, ssem):
        c = lax.axis_index("c")
        s = lax.axis_index("s")
        _zero_fill(zb_v, ZR, width)
        r0 = s * rows_per_tile
        for t in range(rows_per_tile // ZR):
            pltpu.sync_copy(zb_v, acc_sh.at[pl.ds(r0 + t * ZR, ZR)])
        plsc.subcore_barrier()

        ngath = jnp.where(c == 0, ch_fast // NB, ch_slow // NB)
        base0 = jnp.where(c == 0, s * ch_fast, NS * ch_fast + s * ch_slow)

        def body(t, carry):
            row0 = base0 + t * NB
            pltpu.sync_copy(src_hbm.at[pl.ds(row0, NB)], src_v)
            pltpu.sync_copy(dst_hbm.at[pl.ds(row0, NB)], dst_v)
            if shift:
                for b in range(NB):
                    for j in range(K // 16):
                        sl = pl.ds(j * 16, 16)
                        src_v[b, sl] = lax.shift_right_logical(src_v[b, sl], 1)
                        dst_v[b, sl] = lax.shift_right_logical(dst_v[b, sl], 1)
            gds = [pltpu.async_copy(tab_hbm.at[src_v.at[b]], rows_v.at[b], gsem)
                   for b in range(NB)]
            sds = []
            for b in range(NB):
                gds[b].wait()
                sds.append(pltpu.async_copy(rows_v.at[b], acc_sh.at[dst_v.at[b]],
                                            ssem, add=True))
            for b in range(NB):
                sds[b].wait()
            return carry

        lax.fori_loop(0, ngath, body, 0)
        plsc.subcore_barrier()

        @pl.when(c == 0)
        def _():
            pltpu.sync_copy(acc_sh.at[pl.ds(r0, rows_per_tile)],
                            out_hbm.at[pl.ds(r0, rows_per_tile)])

        if split:
            @pl.when(c == 1)
            def _():
                pltpu.sync_copy(acc_sh.at[pl.ds(r0, rows_per_tile)],
                                out_hbm.at[pl.ds(n_rows + r0, rows_per_tile)])

    return seg_sum


@functools.cache
def _get_sc_layer1():
    return _make_sc_layer1()


@functools.cache
def _get_sc_seg(n_rows, width, shift, ch_fast, ch_slow):
    return _make_sc_seg(n_rows, width, shift, ch_fast, ch_slow)


def _sc_layer1(table, src, dst, ones):
    return _get_sc_layer1()(table, src, dst, ones)


def _sc_seg(table, src, dst, n_rows, width, shift):
    ch_fast, ch_slow = (96, 64) if shift else (CPT, 0)
    return _get_sc_seg(n_rows, width, shift, ch_fast, ch_slow)(table, src, dst)


def _p1_body(x_ref, wl_ref, wr_ref, y1_ref, z1_ref):
    xb = x_ref[...]
    y1_ref[...] = jnp.dot(xb, wl_ref[...], preferred_element_type=jnp.float32)
    z1_ref[...] = jnp.dot(xb, wr_ref[...], preferred_element_type=jnp.float32)


def _p1(x_pad, W_in_l, W_in_r):
    return pl.pallas_call(
        _p1_body,
        grid=(N_PAD // RB,),
        in_specs=[
            pl.BlockSpec((RB, 128), lambda i: (i, 0)),
            pl.BlockSpec((128, 64), lambda i: (0, 0)),
            pl.BlockSpec((128, 64), lambda i: (0, 0)),
        ],
        out_specs=[
            pl.BlockSpec((RB, 64), lambda i: (i, 0)),
            pl.BlockSpec((RB, 64), lambda i: (i, 0)),
        ],
        out_shape=[
            jax.ShapeDtypeStruct((N_PAD, 64), jnp.float32),
            jax.ShapeDtypeStruct((N_PAD, 64), jnp.float32),
        ],
    )(x_pad, W_in_l, W_in_r)


def _p2_body(s1_ref, c1_ref, z1_ref, bi_ref, whl_ref, whr_ref, y2_ref, z2_ref):
    i = pl.program_id(0)
    cnt = jnp.maximum(c1_ref[:, :1], 1.0)
    h = s1_ref[...] / cnt + z1_ref[...] + bi_ref[...]
    h = jnp.maximum(h, 0.0)
    nrm = jnp.maximum(jnp.sqrt(jnp.sum(h * h, axis=1, keepdims=True)), 1e-12)
    h = h / nrm
    rows = i * RB + lax.broadcasted_iota(jnp.int32, (RB, 1), 0)
    h = jnp.where(rows < N, h, 0.0)
    y2_ref[...] = jnp.dot(h, whl_ref[...], preferred_element_type=jnp.float32)
    z2_ref[...] = jnp.dot(h, whr_ref[...], preferred_element_type=jnp.float32)


def _p2(s1, c1, z1, bi, W_h_l, W_h_r):
    return pl.pallas_call(
        _p2_body,
        grid=(N_PAD // RB,),
        in_specs=[
            pl.BlockSpec((RB, 64), lambda i: (i, 0)),
            pl.BlockSpec((RB, CW), lambda i: (i, 0)),
            pl.BlockSpec((RB, 64), lambda i: (i, 0)),
            pl.BlockSpec((1, 64), lambda i: (0, 0)),
            pl.BlockSpec((64, 64), lambda i: (0, 0)),
            pl.BlockSpec((64, 64), lambda i: (0, 0)),
        ],
        out_specs=[
            pl.BlockSpec((RB, 64), lambda i: (i, 0)),
            pl.BlockSpec((RB, 64), lambda i: (i, 0)),
        ],
        out_shape=[
            jax.ShapeDtypeStruct((N_PAD, 64), jnp.float32),
            jax.ShapeDtypeStruct((N_PAD, 64), jnp.float32),
        ],
    )(s1, c1, z1, bi, W_h_l, W_h_r)


def _p3_body(s2_ref, c1_ref, z2_ref, bh_ref, wol_ref, wor_ref, bo_ref,
             y3_ref, z3_ref):
    i = pl.program_id(0)
    cnt = jnp.maximum(c1_ref[:, :1], 1.0)
    h = jnp.maximum(s2_ref[...] / cnt + z2_ref[...] + bh_ref[...], 0.0)
    nrm = jnp.maximum(jnp.sqrt(jnp.sum(h * h, axis=1, keepdims=True)), 1e-12)
    h = h / nrm
    rows = i * RB + lax.broadcasted_iota(jnp.int32, (RB, 1), 0)
    h = jnp.where(rows < N, h, 0.0)
    rj = lax.broadcasted_iota(jnp.int32, (RB // 2, RB), 0)
    ci = lax.broadcasted_iota(jnp.int32, (RB // 2, RB), 1)
    pair = jnp.where(lax.shift_right_logical(ci, 1) == rj, 0.5, 0.0)
    x2 = jnp.dot(pair, h, preferred_element_type=jnp.float32)
    gc = i * (RB // 2) + lax.broadcasted_iota(jnp.int32, (RB // 2, 1), 0)
    cmask = gc < C_REAL
    y3 = jnp.dot(x2, wol_ref[...], preferred_element_type=jnp.float32)
    y3_ref[...] = jnp.where(cmask, y3, 0.0)
    z3_ref[...] = (jnp.dot(x2, wor_ref[...], preferred_element_type=jnp.float32)
                   + bo_ref[...])


def _p3(s2, c1, z2, bh, Wl3, Wr3, bo):
    return pl.pallas_call(
        _p3_body,
        grid=(N_PAD // RB,),
        in_specs=[
            pl.BlockSpec((RB, 64), lambda i: (i, 0)),
            pl.BlockSpec((RB, CW), lambda i: (i, 0)),
            pl.BlockSpec((RB, 64), lambda i: (i, 0)),
            pl.BlockSpec((1, 64), lambda i: (0, 0)),
            pl.BlockSpec((64, 16), lambda i: (0, 0)),
            pl.BlockSpec((64, 16), lambda i: (0, 0)),
            pl.BlockSpec((1, 16), lambda i: (0, 0)),
        ],
        out_specs=[
            pl.BlockSpec((RB // 2, 16), lambda i: (i, 0)),
            pl.BlockSpec((RB // 2, 16), lambda i: (i, 0)),
        ],
        out_shape=[
            jax.ShapeDtypeStruct((C_PAD, 16), jnp.float32),
            jax.ShapeDtypeStruct((C_PAD, 16), jnp.float32),
        ],
    )(s2, c1, z2, bh, Wl3, Wr3, bo)


def _p4_body(s3a_ref, s3b_ref, c2_ref, z3_ref, b2_ref, out_ref):
    cnt2 = jnp.maximum(c2_ref[:, :1], 1.0)
    o = (s3a_ref[...] + s3b_ref[...]) / cnt2 + z3_ref[...]
    col16 = lax.broadcasted_iota(jnp.int32, (1, 16), 1)
    cm = col16 < 10
    o = jnp.where(cm, o, 0.0)
    nrm = jnp.maximum(jnp.sqrt(jnp.sum(o * o, axis=1, keepdims=True)), 1e-12)
    o = o / nrm
    b2 = b2_ref[...]
    col8 = lax.broadcasted_iota(jnp.int32, (1, BATCHES), 1)
    onehot = jnp.where(b2 == col8, 1.0, 0.0)
    dn = (((0,), (0,)), ((), ()))
    gs = lax.dot_general(onehot, o, dn, preferred_element_type=jnp.float32)
    gcnt = lax.dot_general(onehot, jnp.ones_like(o), dn,
                           preferred_element_type=jnp.float32)
    out = gs / jnp.maximum(gcnt, 1.0)
    neg = jnp.where(cm, out, -1e30)
    mx = jnp.max(neg, axis=1, keepdims=True)
    e = jnp.where(cm, jnp.exp(out - mx), 0.0)
    lse = jnp.log(jnp.sum(e, axis=1, keepdims=True))
    out_ref[...] = out - mx - lse


def _p4(s3, c2, z3, batch2):
    return pl.pallas_call(
        _p4_body,
        grid=(1,),
        in_specs=[
            pl.BlockSpec((C_PAD, 16), lambda i: (0, 0)),
            pl.BlockSpec((C_PAD, 16), lambda i: (1, 0)),
            pl.BlockSpec((C_PAD, CW), lambda i: (0, 0)),
            pl.BlockSpec((C_PAD, 16), lambda i: (0, 0)),
            pl.BlockSpec((C_PAD, 1), lambda i: (0, 0)),
        ],
        out_specs=pl.BlockSpec((BATCHES, 16), lambda i: (0, 0)),
        out_shape=jax.ShapeDtypeStruct((BATCHES, 16), jnp.float32),
    )(s3, s3, c2, z3, batch2)


def kernel(x, edge_index, batch, W_in_l, W_in_r, b_in, W_h_l, W_h_r, b_h,
           W_out_l, W_out_r, b_out):
    pad = jnp.full((E_PAD - E,), P_IDX, jnp.int32)
    src = jnp.concatenate([edge_index[0], pad]).reshape(E_PAD // K, K)
    dst = jnp.concatenate([edge_index[1], pad]).reshape(E_PAD // K, K)
    x_pad = jnp.pad(x, ((0, N_PAD - N), (0, 0)))
    batch2 = jnp.concatenate(
        [batch[0::2], jnp.full((C_PAD - C_REAL,), BATCHES, jnp.int32)]
    ).reshape(C_PAD, 1)
    Wl3 = jnp.pad(W_out_l, ((0, 0), (0, 6)))
    Wr3 = jnp.pad(W_out_r, ((0, 0), (0, 6)))
    bo = jnp.pad(b_out, (0, 6)).reshape(1, 16)
    bi = b_in.reshape(1, 64)
    bh = b_h.reshape(1, 64)

    ones = jnp.ones((K, CW), jnp.float32)
    y1, z1 = _p1(x_pad, W_in_l, W_in_r)
    s1, c1, c2 = _sc_layer1(y1, src, dst, ones)
    y2, z2 = _p2(s1, c1, z1, bi, W_h_l, W_h_r)
    s2 = _sc_seg(y2, src, dst, N_PAD, 64, False)
    y3, z3 = _p3(s2, c1, z2, bh, Wl3, Wr3, bo)
    s3 = _sc_seg(y3, src, dst, C_PAD, 16, True)
    out = _p4(s3, c2, z3, batch2)
    return out[:, :10]
